# Initial kernel scaffold; baseline (speedup 1.0000x reference)
import jax
import jax.numpy as jnp
from jax.experimental import pallas as pl


def kernel(x, edge_index, edge_attr, batchs, Wq, bq, Wk, bk, Wv, bv, We, Wskip, bskip, linl_w, linl_b, fc_w, fc_b):
    raise NotImplementedError("write your pallas kernel here")


# speedup vs baseline: 4.5896x; 4.5896x over previous
import functools

import jax
import jax.numpy as jnp
from jax import lax
from jax.experimental import pallas as pl
from jax.experimental.pallas import tpu as pltpu
from jax.experimental.pallas import tpu_sc as plsc

_N = 10000
_E = 320000
_D = 128
_ED = 16
_G = 64

_NTILES = 32
_NB = 320
_NPAD = _NTILES * _NB
_CH = 128
_EPAD = _E + _CH
_RB = 512
_NEG_INF = float("-inf")


def _mesh():
    return plsc.VectorSubcoreMesh(core_axis_name="c", subcore_axis_name="s")


def _wid():
    return lax.axis_index("s") * 2 + lax.axis_index("c")


def _sc_prep(edge_attr, perm):
    ept = _E // _NTILES
    nch = ept // _CH
    rem = ept - nch * _CH

    @functools.partial(
        pl.kernel,
        out_type=jax.ShapeDtypeStruct((_EPAD, _ED), jnp.float32),
        mesh=_mesh(),
        compiler_params=pltpu.CompilerParams(
            needs_layout_passes=False, use_tc_tiling_on_sc=False),
        scratch_types=[
            pltpu.VMEM((_CH,), jnp.int32),
            pltpu.VMEM((_CH, _ED), jnp.float32),
            pltpu.SemaphoreType.DMA,
        ],
    )
    def prep(ea_h, perm_h, eas_h, permv, eav, sem):
        wid = _wid()
        e0 = wid * ept

        def chunk_body(i, _):
            base = e0 + i * _CH
            pltpu.sync_copy(perm_h.at[pl.ds(base, _CH)], permv)
            pltpu.async_copy(ea_h.at[permv], eav, sem).wait()
            pltpu.sync_copy(eav, eas_h.at[pl.ds(base, _CH)])
            return 0
        lax.fori_loop(0, nch, chunk_body, 0)

        if rem:
            rbase = e0 + nch * _CH
            def rem_body(t, _):
                b = rbase + t * 16
                pltpu.sync_copy(perm_h.at[pl.ds(b, 16)],
                                permv.at[pl.ds(0, 16)])
                pltpu.async_copy(ea_h.at[permv.at[pl.ds(0, 16)]],
                                 eav.at[pl.ds(0, 16)], sem).wait()
                pltpu.sync_copy(eav.at[pl.ds(0, 16)],
                                eas_h.at[pl.ds(b, 16)])
                return 0
            lax.fori_loop(0, rem // 16, rem_body, 0)

        @pl.when(wid == 0)
        def _():
            for r in range(_CH):
                eav[r] = jnp.zeros((_ED,), jnp.float32)
            pltpu.sync_copy(eav, eas_h.at[pl.ds(_E, _CH)])

    return prep(edge_attr, perm)


def _sc_edges(q, k, v, qe, src_s, dst_s, ea_s, offsp):
    inv_sqrt_d = 1.0 / (_D ** 0.5)

    @functools.partial(
        pl.kernel,
        out_type=(
            jax.ShapeDtypeStruct((_NPAD, _D), jnp.float32),
            jax.ShapeDtypeStruct((_NPAD, _ED), jnp.float32),
            jax.ShapeDtypeStruct((_NTILES, _E + _CH), jnp.float32),
        ),
        mesh=_mesh(),
        compiler_params=pltpu.CompilerParams(
            needs_layout_passes=False, use_tc_tiling_on_sc=False),
        scratch_types=[
            pltpu.VMEM((_NB, _D), jnp.float32),
            pltpu.VMEM((_NB, _ED), jnp.float32),
            pltpu.VMEM((_NB, _D), jnp.float32),
            pltpu.VMEM((_NB, _ED), jnp.float32),
            pltpu.VMEM((_NB + 24,), jnp.int32),
            pltpu.VMEM((_NB + 16,), jnp.float32),
            pltpu.VMEM((_NB + 16,), jnp.float32),
            pltpu.VMEM((_NB,), jnp.float32),
            pltpu.VMEM((_CH,), jnp.int32),
            pltpu.VMEM((_CH + 16,), jnp.int32),
            pltpu.VMEM((_CH, _D), jnp.float32),
            pltpu.VMEM((_CH, _ED), jnp.float32),
            pltpu.VMEM((_CH + 16,), jnp.float32),
            pltpu.VMEM((_CH + 16,), jnp.float32),
            pltpu.SemaphoreType.DMA,
        ],
    )
    def edges(q_h, k_h, v_h, qe_h, srcs_h, dsts_h, eas_h, offs_h,
              agg_h, agg16_h, ascr_h,
              qloc, qeloc, aggbuf, agg16buf, offsloc, m_s, s_s, rdiv,
              idxv, dstv, kvbuf, eabuf, alphab, attnb, sem):
        wid = _wid()
        n0 = wid * _NB
        iota = lax.broadcasted_iota(jnp.int32, (16,), 0)

        def sload(ref, i):
            return ref[pl.ds(i, 16)][0]

        def sstore(ref, i, val):
            w = ref[pl.ds(i, 16)]
            ref[pl.ds(i, 16)] = jnp.where(iota == 0, val, w)

        pltpu.sync_copy(q_h.at[pl.ds(n0, _NB)], qloc)
        pltpu.sync_copy(qe_h.at[pl.ds(n0, _NB)], qeloc)
        pltpu.sync_copy(offs_h.at[pl.ds(n0, _NB + 8)],
                        offsloc.at[pl.ds(0, _NB + 8)])
        e0 = (sload(offsloc, 0) // 8) * 8
        e1 = sload(offsloc, _NB)
        ecnt = e1 - e0
        nchunks = (ecnt + _CH - 1) // _CH

        def init_body(r, _):
            for j in range(_D // 16):
                aggbuf[r, pl.ds(16 * j, 16)] = jnp.zeros((16,), jnp.float32)
            agg16buf[r] = jnp.zeros((_ED,), jnp.float32)
            return 0
        lax.fori_loop(0, _NB, init_body, 0)
        for t in range(_NB // 16):
            m_s[pl.ds(16 * t, 16)] = jnp.full((16,), _NEG_INF, jnp.float32)
            s_s[pl.ds(16 * t, 16)] = jnp.zeros((16,), jnp.float32)

        def seg_pieces(lc, chv, per_node):
            nlo = jnp.maximum(sload(dstv, 0), n0)
            last = jnp.maximum(chv - 1, 0)
            nhi = jnp.minimum(sload(dstv, last), n0 + _NB - 1)

            def node_body(n, _):
                nloc = n - n0
                w = offsloc[pl.ds(nloc, 16)]
                la = w[0] - e0
                lb = w[1] - e0
                lo = jnp.maximum(la, lc)
                hi = jnp.minimum(lb, lc + chv)
                per_node(nloc, lo - lc, hi - lo)
                return 0
            lax.fori_loop(nlo, nhi + 1, node_body, 0)

        def passA(i, _):
            lc = i * _CH
            chv = jnp.minimum(ecnt - lc, _CH)
            pltpu.sync_copy(srcs_h.at[pl.ds(e0 + lc, _CH)], idxv)
            pltpu.sync_copy(dsts_h.at[pl.ds(e0 + lc, _CH)],
                            dstv.at[pl.ds(0, _CH)])
            pltpu.sync_copy(eas_h.at[pl.ds(e0 + lc, _CH)], eabuf)
            pltpu.async_copy(k_h.at[idxv], kvbuf, sem).wait()

            def group_body(g, _):
                def lane_body(t, vec):
                    e = g * 16 + t
                    dc = jnp.clip(sload(dstv, e) - n0, 0, _NB - 1)
                    acc = qeloc[dc] * eabuf[e]
                    for j in range(_D // 16):
                        acc = acc + (qloc[dc, pl.ds(16 * j, 16)]
                                     * kvbuf[e, pl.ds(16 * j, 16)])
                    a = jnp.sum(acc) * inv_sqrt_d
                    return jnp.where(iota == t, a, vec)
                vec = lax.fori_loop(0, 16, lane_body,
                                    jnp.zeros((16,), jnp.float32))
                alphab[pl.ds(g * 16, 16)] = vec
                return 0
            lax.fori_loop(0, _CH // 16, group_body, 0)
            pltpu.sync_copy(alphab.at[pl.ds(0, _CH)],
                            ascr_h.at[wid, pl.ds(lc, _CH)])

            def upd_max(nloc, base, nt):
                def vbody(t, cm):
                    av = alphab[pl.ds(base + t * 16, 16)]
                    msk = iota < (nt - t * 16)
                    return jnp.maximum(cm, jnp.where(msk, av, _NEG_INF))
                cm = lax.fori_loop(0, (nt + 15) // 16, vbody,
                                   jnp.full((16,), _NEG_INF, jnp.float32))
                sstore(m_s, nloc,
                       jnp.maximum(sload(m_s, nloc), jnp.max(cm)))
            seg_pieces(lc, chv, upd_max)
            return 0
        lax.fori_loop(0, nchunks, passA, 0)

        def passB(i, _):
            lc = i * _CH
            chv = jnp.minimum(ecnt - lc, _CH)
            pltpu.sync_copy(dsts_h.at[pl.ds(e0 + lc, _CH)],
                            dstv.at[pl.ds(0, _CH)])
            pltpu.sync_copy(ascr_h.at[wid, pl.ds(lc, _CH)],
                            alphab.at[pl.ds(0, _CH)])

            def upd_sum(nloc, base, nt):
                mn = jnp.full((16,), sload(m_s, nloc))
                def vbody(t, cs):
                    av = alphab[pl.ds(base + t * 16, 16)]
                    msk = iota < (nt - t * 16)
                    return cs + jnp.where(msk, jnp.exp(av - mn), 0.0)
                cs = lax.fori_loop(0, (nt + 15) // 16, vbody,
                                   jnp.zeros((16,), jnp.float32))
                sstore(s_s, nloc, sload(s_s, nloc) + jnp.sum(cs))
            seg_pieces(lc, chv, upd_sum)
            return 0
        lax.fori_loop(0, nchunks, passB, 0)

        for t in range(_NB // 16):
            sl = pl.ds(16 * t, 16)
            rdiv[sl] = 1.0 / (s_s[sl] + 1e-16)

        def passC(i, _):
            lc = i * _CH
            pltpu.sync_copy(srcs_h.at[pl.ds(e0 + lc, _CH)], idxv)
            pltpu.sync_copy(dsts_h.at[pl.ds(e0 + lc, _CH)],
                            dstv.at[pl.ds(0, _CH)])
            pltpu.sync_copy(eas_h.at[pl.ds(e0 + lc, _CH)], eabuf)
            pltpu.sync_copy(ascr_h.at[wid, pl.ds(lc, _CH)],
                            alphab.at[pl.ds(0, _CH)])
            pltpu.async_copy(v_h.at[idxv], kvbuf, sem).wait()

            for t in range(_CH // 16):
                sl = pl.ds(16 * t, 16)
                av = alphab[sl]
                dv = dstv[sl]
                dc16 = jnp.clip(dv - n0, 0, _NB - 1)
                mg = plsc.load_gather(m_s, [dc16])
                rg = plsc.load_gather(rdiv, [dc16])
                valid = (((lc + t * 16 + iota) < ecnt)
                         & (dv >= n0) & (dv < n0 + _NB))
                attnb[sl] = jnp.where(valid, jnp.exp(av - mg) * rg, 0.0)

            def edge_body(e, _):
                dc = jnp.clip(sload(dstv, e) - n0, 0, _NB - 1)
                aw = jnp.full((16,), sload(attnb, e))
                for j in range(_D // 16):
                    plsc.addupdate(aggbuf.at[dc, pl.ds(16 * j, 16)],
                                   aw * kvbuf[e, pl.ds(16 * j, 16)])
                plsc.addupdate(agg16buf.at[dc], aw * eabuf[e])
                return 0
            lax.fori_loop(0, _CH, edge_body, 0, unroll=2)
            return 0
        lax.fori_loop(0, nchunks, passC, 0)

        pltpu.sync_copy(aggbuf, agg_h.at[pl.ds(n0, _NB)])
        pltpu.sync_copy(agg16buf, agg16_h.at[pl.ds(n0, _NB)])

    agg128, agg16, _unused = edges(q, k, v, qe, src_s, dst_s, ea_s, offsp)
    return agg128, agg16


def _tc_qkv(h, Wcat, bcat, WeT):
    grid = _NPAD // _RB

    def body(h_ref, w_ref, b_ref, wet_ref, q_ref, k_ref, v_ref, s_ref, qe_ref):
        cat = jnp.dot(h_ref[...], w_ref[...],
                      preferred_element_type=jnp.float32) + b_ref[...]
        q = cat[:, 0:_D]
        q_ref[...] = q
        k_ref[...] = cat[:, _D:2 * _D]
        v_ref[...] = cat[:, 2 * _D:3 * _D]
        s_ref[...] = cat[:, 3 * _D:4 * _D]
        qe_ref[...] = jnp.dot(q, wet_ref[...],
                              preferred_element_type=jnp.float32)

    nd = jax.ShapeDtypeStruct((_NPAD, _D), jnp.float32)
    return pl.pallas_call(
        body,
        grid=(grid,),
        in_specs=[
            pl.BlockSpec((_RB, _D), lambda i: (i, 0)),
            pl.BlockSpec((_D, 4 * _D), lambda i: (0, 0)),
            pl.BlockSpec((1, 4 * _D), lambda i: (0, 0)),
            pl.BlockSpec((_D, _ED), lambda i: (0, 0)),
        ],
        out_specs=[
            pl.BlockSpec((_RB, _D), lambda i: (i, 0)),
            pl.BlockSpec((_RB, _D), lambda i: (i, 0)),
            pl.BlockSpec((_RB, _D), lambda i: (i, 0)),
            pl.BlockSpec((_RB, _D), lambda i: (i, 0)),
            pl.BlockSpec((_RB, _ED), lambda i: (i, 0)),
        ],
        out_shape=[nd, nd, nd, nd,
                   jax.ShapeDtypeStruct((_NPAD, _ED), jnp.float32)],
    )(h, Wcat, bcat, WeT)


def _tc_combine(agg128, agg16, skipc, We, relu):
    grid = _NPAD // _RB

    def body(a_ref, a16_ref, s_ref, we_ref, o_ref):
        o = a_ref[...] + s_ref[...] + jnp.dot(
            a16_ref[...], we_ref[...], preferred_element_type=jnp.float32)
        if relu:
            o = jnp.maximum(o, 0.0)
        o_ref[...] = o

    return pl.pallas_call(
        body,
        grid=(grid,),
        in_specs=[
            pl.BlockSpec((_RB, _D), lambda i: (i, 0)),
            pl.BlockSpec((_RB, _ED), lambda i: (i, 0)),
            pl.BlockSpec((_RB, _D), lambda i: (i, 0)),
            pl.BlockSpec((_ED, _D), lambda i: (0, 0)),
        ],
        out_specs=pl.BlockSpec((_RB, _D), lambda i: (i, 0)),
        out_shape=jax.ShapeDtypeStruct((_NPAD, _D), jnp.float32),
    )(agg128, agg16, skipc, We)


def _tc_pool(h, batch3, linl_w, linl_b, fc_w, fc_b):
    grid = _NPAD // _RB

    def body(h_ref, b_ref, lw_ref, lb_ref, fw_ref, fb_ref, o_ref, pooled, cnt):
        i = pl.program_id(0)

        @pl.when(i == 0)
        def _():
            pooled[...] = jnp.zeros_like(pooled)
            cnt[...] = jnp.zeros_like(cnt)

        bb = b_ref[0]
        gi = lax.broadcasted_iota(jnp.int32, (_G, _RB), 0)
        oh = (jnp.broadcast_to(bb, (_G, _RB)) == gi).astype(jnp.float32)
        pooled[...] += jnp.dot(oh, h_ref[...],
                               preferred_element_type=jnp.float32)
        cnt[...] += jnp.sum(oh, axis=1, keepdims=True)

        @pl.when(i == grid - 1)
        def _():
            pm = pooled[...] / jnp.maximum(cnt[...], 1.0)
            t = jnp.maximum(
                jnp.dot(pm, lw_ref[...], preferred_element_type=jnp.float32)
                + lb_ref[...], 0.0)
            o_ref[...] = jnp.dot(t, fw_ref[...],
                                 preferred_element_type=jnp.float32) + fb_ref[...]

    return pl.pallas_call(
        body,
        grid=(grid,),
        in_specs=[
            pl.BlockSpec((_RB, _D), lambda i: (i, 0)),
            pl.BlockSpec((1, 1, _RB), lambda i: (i, 0, 0)),
            pl.BlockSpec((_D, _D), lambda i: (0, 0)),
            pl.BlockSpec((1, _D), lambda i: (0, 0)),
            pl.BlockSpec((_D, 1), lambda i: (0, 0)),
            pl.BlockSpec((1, 1), lambda i: (0, 0)),
        ],
        out_specs=pl.BlockSpec((_G, 1), lambda i: (0, 0)),
        out_shape=jax.ShapeDtypeStruct((_G, 1), jnp.float32),
        scratch_shapes=[
            pltpu.VMEM((_G, _D), jnp.float32),
            pltpu.VMEM((_G, 1), jnp.float32),
        ],
    )(h, batch3, linl_w, linl_b, fc_w, fc_b)


@jax.jit
def kernel(x, edge_index, edge_attr, batchs, Wq, bq, Wk, bk, Wv, bv, We,
           Wskip, bskip, linl_w, linl_b, fc_w, fc_b):
    src = edge_index[0]
    dst = edge_index[1]

    perm = jnp.argsort(dst).astype(jnp.int32)
    dst_sorted = jnp.sort(dst)
    src_s = jnp.concatenate(
        [src[perm], jnp.zeros((_EPAD - _E,), jnp.int32)])
    dst_s = jnp.concatenate(
        [dst_sorted, jnp.full((_EPAD - _E,), _NPAD, jnp.int32)])
    ea_s = _sc_prep(edge_attr, perm)
    offsp = jnp.searchsorted(
        dst_sorted,
        jnp.arange(_NPAD + 16, dtype=jnp.int32), side="left").astype(jnp.int32)

    h = jnp.concatenate(
        [x, jnp.zeros((_NPAD - _N, _D), jnp.float32)], axis=0)

    Wcat = jnp.concatenate([Wq, Wk, Wv, Wskip], axis=2)
    bcat = jnp.concatenate([bq, bk, bv, bskip], axis=1)
    WeT = jnp.swapaxes(We, 1, 2)

    for l in range(4):
        qn, kn, vn, skipc, qe = _tc_qkv(
            h, Wcat[l], bcat[l][None, :], WeT[l])
        agg128, agg16 = _sc_edges(qn, kn, vn, qe, src_s, dst_s, ea_s, offsp)
        h = _tc_combine(agg128, agg16, skipc, We[l], relu=(l < 3))

    batch3 = jnp.concatenate(
        [batchs, jnp.full((_NPAD - _N,), _G, jnp.int32)]).reshape(
            _NPAD // _RB, 1, _RB)
    return _tc_pool(h, batch3, linl_w, linl_b[None, :], fc_w,
                    jnp.reshape(fc_b, (1, 1)))
